# Initial kernel scaffold; baseline (speedup 1.0000x reference)
import functools

import jax
import jax.numpy as jnp
from jax import lax
from jax.experimental import pallas as pl
from jax.experimental.pallas import tpu as pltpu
from jax.experimental.pallas import tpu_sc as plsc

N_NODES = 10000
N_EDGES = 320000
D_IN = 128
D_HID = 64

NC = 2
NS = 16
NW = NC * NS
EPW = N_EDGES // NW
CH = 80
NCHUNK = EPW // CH
RPT = N_NODES // NS

_sc_mesh = plsc.VectorSubcoreMesh(core_axis_name="c", subcore_axis_name="s")


@functools.partial(
    pl.kernel,
    out_type=jax.ShapeDtypeStruct((NC, N_NODES, D_HID), jnp.float32),
    mesh=_sc_mesh,
    scratch_types=[
        pltpu.VMEM((CH,), jnp.int32),
        pltpu.VMEM((CH,), jnp.int32),
        pltpu.VMEM((CH, D_HID), jnp.float32),
        pltpu.VMEM_SHARED((N_NODES, D_HID), jnp.float32),
        pltpu.SemaphoreType.DMA,
    ],
)
def _sc_aggregate(y_hbm, src_hbm, dst_hbm, zeros_hbm, out_hbm,
                  sidx, didx, rows, acc, sem):
    cid = lax.axis_index("c")
    sid = lax.axis_index("s")
    pltpu.sync_copy(zeros_hbm, acc.at[pl.ds(sid * RPT, RPT)])
    plsc.subcore_barrier()

    wid = cid * NS + sid
    ebase = wid * EPW

    def body(g, carry):
        off = pl.multiple_of(ebase + g * CH, 8)
        pltpu.sync_copy(src_hbm.at[pl.ds(off, CH)], sidx)
        pltpu.sync_copy(dst_hbm.at[pl.ds(off, CH)], didx)
        pltpu.async_copy(y_hbm.at[sidx], rows, sem).wait()
        pltpu.sync_copy(rows, acc.at[didx], add=True)
        return carry

    lax.fori_loop(0, NCHUNK, body, 0)
    plsc.subcore_barrier()
    pltpu.sync_copy(acc.at[pl.ds(sid * RPT, RPT)],
                    out_hbm.at[cid, pl.ds(sid * RPT, RPT)])


def _mm1_body(x_ref, w_ref, o_ref):
    o_ref[...] = jnp.dot(x_ref[...], w_ref[...],
                         preferred_element_type=jnp.float32)


def _mid_body(y1_ref, agg_ref, eps_ref, b11_ref, w12_ref, b12_ref, w21_ref,
              emb_ref, y2_ref):
    pre = ((1.0 + eps_ref[0, 0]) * y1_ref[...]
           + agg_ref[0] + agg_ref[1] + b11_ref[...])
    h1 = jnp.maximum(pre, 0.0)
    emb = jnp.maximum(
        jnp.dot(h1, w12_ref[...], preferred_element_type=jnp.float32)
        + b12_ref[...], 0.0)
    emb_ref[...] = emb
    y2_ref[...] = jnp.dot(emb, w21_ref[...],
                          preferred_element_type=jnp.float32)


def _final_body(y2_ref, agg_ref, eps_ref, b21_ref, w22_ref, b22_ref, o_ref):
    pre = ((1.0 + eps_ref[0, 0]) * y2_ref[...]
           + agg_ref[0] + agg_ref[1] + b21_ref[...])
    a = jnp.maximum(pre, 0.0)
    h2 = (jnp.dot(a, w22_ref[...], preferred_element_type=jnp.float32)
          + b22_ref[...])
    m = jnp.max(h2, axis=1, keepdims=True)
    lse = jnp.log(jnp.sum(jnp.exp(h2 - m), axis=1, keepdims=True)) + m
    o_ref[...] = h2 - lse


def kernel(x, edge_index, eps1, W11, b11, W12, b12, eps2, W21, b21, W22, b22):
    src = edge_index[0].astype(jnp.int32)
    dst = edge_index[1].astype(jnp.int32)
    zeros = jnp.zeros((RPT, D_HID), jnp.float32)

    y1 = pl.pallas_call(
        _mm1_body,
        out_shape=jax.ShapeDtypeStruct((N_NODES, D_HID), jnp.float32),
    )(x, W11)

    agg1 = _sc_aggregate(y1, src, dst, zeros)

    emb, y2 = pl.pallas_call(
        _mid_body,
        out_shape=(jax.ShapeDtypeStruct((N_NODES, D_HID), jnp.float32),
                   jax.ShapeDtypeStruct((N_NODES, D_HID), jnp.float32)),
    )(y1, agg1, eps1.reshape(1, 1), b11.reshape(1, D_HID), W12,
      b12.reshape(1, D_HID), W21)

    agg2 = _sc_aggregate(y2, src, dst, zeros)

    out = pl.pallas_call(
        _final_body,
        out_shape=jax.ShapeDtypeStruct((N_NODES, D_HID), jnp.float32),
    )(y2, agg2, eps2.reshape(1, 1), b21.reshape(1, D_HID), W22,
      b22.reshape(1, D_HID))

    return (out, emb)


# speedup vs baseline: 5.6687x; 5.6687x over previous
import functools

import jax
import jax.numpy as jnp
from jax import lax
from jax.experimental import pallas as pl
from jax.experimental.pallas import tpu as pltpu
from jax.experimental.pallas import tpu_sc as plsc

N_NODES = 10000
N_EDGES = 320000
D_IN = 128
D_HID = 64

NC = 2
NS = 16
NW = NC * NS
EPW = N_EDGES // NW
CH = 80
NCHUNK = EPW // CH
N_PAD = 10240
RPT = N_PAD // NS

_sc_mesh = plsc.VectorSubcoreMesh(core_axis_name="c", subcore_axis_name="s")


@functools.partial(
    pl.kernel,
    out_type=jax.ShapeDtypeStruct((NC, N_PAD, D_HID), jnp.float32),
    mesh=_sc_mesh,
    compiler_params=pltpu.CompilerParams(use_tc_tiling_on_sc=False),
    scratch_types=[
        pltpu.VMEM((CH,), jnp.int32),
        pltpu.VMEM((CH,), jnp.int32),
        pltpu.VMEM((CH, D_HID), jnp.float32),
        pltpu.VMEM_SHARED((N_PAD, D_HID), jnp.float32),
        pltpu.SemaphoreType.DMA,
    ],
)
def _sc_aggregate(y_hbm, src_hbm, dst_hbm, zeros_hbm, out_hbm,
                  sidx, didx, rows, acc, sem):
    cid = lax.axis_index("c")
    sid = lax.axis_index("s")
    pltpu.sync_copy(zeros_hbm, acc.at[pl.ds(sid * RPT, RPT)])
    plsc.subcore_barrier()

    wid = cid * NS + sid
    ebase = wid * EPW

    def body(g, carry):
        off = pl.multiple_of(ebase + g * CH, 8)
        pltpu.sync_copy(src_hbm.at[pl.ds(off, CH)], sidx)
        pltpu.sync_copy(dst_hbm.at[pl.ds(off, CH)], didx)
        pltpu.async_copy(y_hbm.at[sidx], rows, sem).wait()
        pltpu.sync_copy(rows, acc.at[didx], add=True)
        return carry

    lax.fori_loop(0, NCHUNK, body, 0)
    plsc.subcore_barrier()
    pltpu.sync_copy(acc.at[pl.ds(sid * RPT, RPT)],
                    out_hbm.at[cid, pl.ds(sid * RPT, RPT)])


def _mm1_body(x_ref, w_ref, o_ref):
    o_ref[...] = jnp.dot(x_ref[...], w_ref[...],
                         preferred_element_type=jnp.float32)


def _mid_body(y1_ref, agg_ref, eps_ref, b11_ref, w12_ref, b12_ref, w21_ref,
              emb_ref, y2_ref):
    pre = ((1.0 + eps_ref[0, 0]) * y1_ref[...]
           + agg_ref[0, :N_NODES] + agg_ref[1, :N_NODES] + b11_ref[...])
    h1 = jnp.maximum(pre, 0.0)
    emb = jnp.maximum(
        jnp.dot(h1, w12_ref[...], preferred_element_type=jnp.float32)
        + b12_ref[...], 0.0)
    emb_ref[...] = emb
    y2_ref[...] = jnp.dot(emb, w21_ref[...],
                          preferred_element_type=jnp.float32)


def _final_body(y2_ref, agg_ref, eps_ref, b21_ref, w22_ref, b22_ref, o_ref):
    pre = ((1.0 + eps_ref[0, 0]) * y2_ref[...]
           + agg_ref[0, :N_NODES] + agg_ref[1, :N_NODES] + b21_ref[...])
    a = jnp.maximum(pre, 0.0)
    h2 = (jnp.dot(a, w22_ref[...], preferred_element_type=jnp.float32)
          + b22_ref[...])
    m = jnp.max(h2, axis=1, keepdims=True)
    lse = jnp.log(jnp.sum(jnp.exp(h2 - m), axis=1, keepdims=True)) + m
    o_ref[...] = h2 - lse


def kernel(x, edge_index, eps1, W11, b11, W12, b12, eps2, W21, b21, W22, b22):
    src = edge_index[0].astype(jnp.int32)
    dst = edge_index[1].astype(jnp.int32)
    zeros = jnp.zeros((RPT, D_HID), jnp.float32)

    y1 = pl.pallas_call(
        _mm1_body,
        out_shape=jax.ShapeDtypeStruct((N_NODES, D_HID), jnp.float32),
    )(x, W11)

    agg1 = _sc_aggregate(y1, src, dst, zeros)

    emb, y2 = pl.pallas_call(
        _mid_body,
        out_shape=(jax.ShapeDtypeStruct((N_NODES, D_HID), jnp.float32),
                   jax.ShapeDtypeStruct((N_NODES, D_HID), jnp.float32)),
    )(y1, agg1, eps1.reshape(1, 1), b11.reshape(1, D_HID), W12,
      b12.reshape(1, D_HID), W21)

    agg2 = _sc_aggregate(y2, src, dst, zeros)

    out = pl.pallas_call(
        _final_body,
        out_shape=jax.ShapeDtypeStruct((N_NODES, D_HID), jnp.float32),
    )(y2, agg2, eps2.reshape(1, 1), b21.reshape(1, D_HID), W22,
      b22.reshape(1, D_HID))

    return (out, emb)


# speedup vs baseline: 11.6500x; 2.0552x over previous
import functools

import jax
import jax.numpy as jnp
from jax import lax
from jax.experimental import pallas as pl
from jax.experimental.pallas import tpu as pltpu
from jax.experimental.pallas import tpu_sc as plsc

N_NODES = 10000
N_EDGES = 320000
D_IN = 128
D_HID = 64

NC = 2
NS = 16
NW = NC * NS
EPW = N_EDGES // NW
CH = 80
NCHUNK = EPW // CH
NPAIR = (NCHUNK - 1) // 2
N_PAD = 10240
RPT = N_PAD // NS

_sc_mesh = plsc.VectorSubcoreMesh(core_axis_name="c", subcore_axis_name="s")


@functools.partial(
    pl.kernel,
    out_type=jax.ShapeDtypeStruct((NC, N_PAD, D_HID), jnp.float32),
    mesh=_sc_mesh,
    compiler_params=pltpu.CompilerParams(use_tc_tiling_on_sc=False),
    scratch_types=[
        pltpu.VMEM((NCHUNK, CH), jnp.int32),
        pltpu.VMEM((NCHUNK, CH), jnp.int32),
        pltpu.VMEM((CH, D_HID), jnp.float32),
        pltpu.VMEM((CH, D_HID), jnp.float32),
        pltpu.VMEM_SHARED((N_PAD, D_HID), jnp.float32),
        pltpu.SemaphoreType.DMA,
        pltpu.SemaphoreType.DMA,
        pltpu.SemaphoreType.DMA,
        pltpu.SemaphoreType.DMA,
    ],
)
def _sc_aggregate(y_hbm, src_hbm, dst_hbm, zeros_hbm, out_hbm,
                  sidx, didx, rows0, rows1, acc, gs0, gs1, ss0, ss1):
    cid = lax.axis_index("c")
    sid = lax.axis_index("s")
    wid = cid * NS + sid

    pltpu.sync_copy(zeros_hbm, acc.at[pl.ds(sid * RPT, RPT)])
    pltpu.sync_copy(src_hbm.at[wid], sidx)
    pltpu.sync_copy(dst_hbm.at[wid], didx)
    plsc.subcore_barrier()

    rows = (rows0, rows1)
    gsem = (gs0, gs1)
    ssem = (ss0, ss1)

    def gather(g, b):
        pltpu.async_copy(y_hbm.at[sidx.at[g]], rows[b], gsem[b])

    def gather_wait(g, b):
        pltpu.make_async_copy(y_hbm.at[sidx.at[g]], rows[b], gsem[b]).wait()

    def scatter(g, b):
        pltpu.async_copy(rows[b], acc.at[didx.at[g]], ssem[b], add=True)

    def scatter_wait(g, b):
        pltpu.make_async_copy(rows[b], acc.at[didx.at[g]], ssem[b]).wait()

    gather(0, 0)
    gather(1, 1)

    def body(i, carry):
        g = 2 * i
        gather_wait(g, 0)
        scatter(g, 0)
        gather_wait(g + 1, 1)
        scatter(g + 1, 1)
        scatter_wait(g, 0)
        gather(g + 2, 0)
        scatter_wait(g + 1, 1)

        @pl.when(i < NPAIR - 1)
        def _():
            gather(g + 3, 1)

        return carry

    lax.fori_loop(0, NPAIR, body, 0)
    g = NCHUNK - 1
    gather_wait(g, 0)
    scatter(g, 0)
    scatter_wait(g, 0)

    plsc.subcore_barrier()
    pltpu.sync_copy(acc.at[pl.ds(sid * RPT, RPT)],
                    out_hbm.at[cid, pl.ds(sid * RPT, RPT)])


def _mm1_body(x_ref, w_ref, o_ref):
    o_ref[...] = jnp.dot(x_ref[...], w_ref[...],
                         preferred_element_type=jnp.float32)


def _mid_body(y1_ref, agg_ref, eps_ref, b11_ref, w12_ref, b12_ref, w21_ref,
              emb_ref, y2_ref):
    pre = ((1.0 + eps_ref[0, 0]) * y1_ref[...]
           + agg_ref[0, :N_NODES] + agg_ref[1, :N_NODES] + b11_ref[...])
    h1 = jnp.maximum(pre, 0.0)
    emb = jnp.maximum(
        jnp.dot(h1, w12_ref[...], preferred_element_type=jnp.float32)
        + b12_ref[...], 0.0)
    emb_ref[...] = emb
    y2_ref[...] = jnp.dot(emb, w21_ref[...],
                          preferred_element_type=jnp.float32)


def _final_body(y2_ref, agg_ref, eps_ref, b21_ref, w22_ref, b22_ref, o_ref):
    pre = ((1.0 + eps_ref[0, 0]) * y2_ref[...]
           + agg_ref[0, :N_NODES] + agg_ref[1, :N_NODES] + b21_ref[...])
    a = jnp.maximum(pre, 0.0)
    h2 = (jnp.dot(a, w22_ref[...], preferred_element_type=jnp.float32)
          + b22_ref[...])
    m = jnp.max(h2, axis=1, keepdims=True)
    lse = jnp.log(jnp.sum(jnp.exp(h2 - m), axis=1, keepdims=True)) + m
    o_ref[...] = h2 - lse


def kernel(x, edge_index, eps1, W11, b11, W12, b12, eps2, W21, b21, W22, b22):
    src = edge_index[0].astype(jnp.int32).reshape(NW, NCHUNK, CH)
    dst = edge_index[1].astype(jnp.int32).reshape(NW, NCHUNK, CH)
    zeros = jnp.zeros((RPT, D_HID), jnp.float32)

    y1 = pl.pallas_call(
        _mm1_body,
        out_shape=jax.ShapeDtypeStruct((N_NODES, D_HID), jnp.float32),
    )(x, W11)

    agg1 = _sc_aggregate(y1, src, dst, zeros)

    emb, y2 = pl.pallas_call(
        _mid_body,
        out_shape=(jax.ShapeDtypeStruct((N_NODES, D_HID), jnp.float32),
                   jax.ShapeDtypeStruct((N_NODES, D_HID), jnp.float32)),
    )(y1, agg1, eps1.reshape(1, 1), b11.reshape(1, D_HID), W12,
      b12.reshape(1, D_HID), W21)

    agg2 = _sc_aggregate(y2, src, dst, zeros)

    out = pl.pallas_call(
        _final_body,
        out_shape=jax.ShapeDtypeStruct((N_NODES, D_HID), jnp.float32),
    )(y2, agg2, eps2.reshape(1, 1), b21.reshape(1, D_HID), W22,
      b22.reshape(1, D_HID))

    return (out, emb)


# speedup vs baseline: 15.0385x; 1.2909x over previous
import functools

import jax
import jax.numpy as jnp
from jax import lax
from jax.experimental import pallas as pl
from jax.experimental.pallas import tpu as pltpu
from jax.experimental.pallas import tpu_sc as plsc

N_NODES = 10000
N_EDGES = 320000
D_IN = 128
D_HID = 64

NC = 2
NS = 16
NW = NC * NS
EPW = N_EDGES // NW
CH = 80
NCHUNK = EPW // CH
NBUF = 4
NGRP = (NCHUNK - 1) // NBUF
N_PAD = 10240
RPT = N_PAD // NS

_sc_mesh = plsc.VectorSubcoreMesh(core_axis_name="c", subcore_axis_name="s")


@functools.partial(
    pl.kernel,
    out_type=jax.ShapeDtypeStruct((NC, N_PAD, D_HID), jnp.float32),
    mesh=_sc_mesh,
    compiler_params=pltpu.CompilerParams(use_tc_tiling_on_sc=False),
    scratch_types=[
        pltpu.VMEM((NCHUNK, CH), jnp.int32),
        pltpu.VMEM((NCHUNK, CH), jnp.int32),
        pltpu.VMEM((CH, D_HID), jnp.float32),
        pltpu.VMEM((CH, D_HID), jnp.float32),
        pltpu.VMEM((CH, D_HID), jnp.float32),
        pltpu.VMEM((CH, D_HID), jnp.float32),
        pltpu.VMEM_SHARED((N_PAD, D_HID), jnp.float32),
        pltpu.SemaphoreType.DMA,
        pltpu.SemaphoreType.DMA,
        pltpu.SemaphoreType.DMA,
        pltpu.SemaphoreType.DMA,
        pltpu.SemaphoreType.DMA,
        pltpu.SemaphoreType.DMA,
        pltpu.SemaphoreType.DMA,
        pltpu.SemaphoreType.DMA,
    ],
)
def _sc_aggregate(y_hbm, src_hbm, dst_hbm, zeros_hbm, out_hbm,
                  sidx, didx, rows0, rows1, rows2, rows3, acc,
                  gs0, gs1, gs2, gs3, ss0, ss1, ss2, ss3):
    cid = lax.axis_index("c")
    sid = lax.axis_index("s")
    wid = cid * NS + sid

    pltpu.sync_copy(zeros_hbm, acc.at[pl.ds(sid * RPT, RPT)])
    pltpu.sync_copy(src_hbm.at[wid], sidx)
    pltpu.sync_copy(dst_hbm.at[wid], didx)
    plsc.subcore_barrier()

    rows = (rows0, rows1, rows2, rows3)
    gsem = (gs0, gs1, gs2, gs3)
    ssem = (ss0, ss1, ss2, ss3)

    def gather(g, b):
        pltpu.async_copy(y_hbm.at[sidx.at[g]], rows[b], gsem[b])

    def gather_wait(g, b):
        pltpu.make_async_copy(y_hbm.at[sidx.at[g]], rows[b], gsem[b]).wait()

    def scatter(g, b):
        pltpu.async_copy(rows[b], acc.at[didx.at[g]], ssem[b], add=True)

    def scatter_wait(g, b):
        pltpu.make_async_copy(rows[b], acc.at[didx.at[g]], ssem[b]).wait()

    for b in range(NBUF):
        gather(b, b)

    def body(i, carry):
        g = NBUF * i
        for b in range(NBUF):
            gather_wait(g + b, b)
            scatter(g + b, b)
        for b in range(NBUF):
            scatter_wait(g + b, b)

            @pl.when(g + b + NBUF < NCHUNK)
            def _():
                gather(g + b + NBUF, b)

        return carry

    lax.fori_loop(0, NGRP, body, 0)
    g = NCHUNK - 1
    gather_wait(g, 0)
    scatter(g, 0)
    scatter_wait(g, 0)

    plsc.subcore_barrier()
    pltpu.sync_copy(acc.at[pl.ds(sid * RPT, RPT)],
                    out_hbm.at[cid, pl.ds(sid * RPT, RPT)])


def _mm1_body(x_ref, w_ref, o_ref):
    o_ref[...] = jnp.dot(x_ref[...], w_ref[...],
                         preferred_element_type=jnp.float32)


def _mid_body(y1_ref, agg_ref, eps_ref, b11_ref, w12_ref, b12_ref, w21_ref,
              emb_ref, y2_ref):
    pre = ((1.0 + eps_ref[0, 0]) * y1_ref[...]
           + agg_ref[0, :N_NODES] + agg_ref[1, :N_NODES] + b11_ref[...])
    h1 = jnp.maximum(pre, 0.0)
    emb = jnp.maximum(
        jnp.dot(h1, w12_ref[...], preferred_element_type=jnp.float32)
        + b12_ref[...], 0.0)
    emb_ref[...] = emb
    y2_ref[...] = jnp.dot(emb, w21_ref[...],
                          preferred_element_type=jnp.float32)


def _final_body(y2_ref, agg_ref, eps_ref, b21_ref, w22_ref, b22_ref, o_ref):
    pre = ((1.0 + eps_ref[0, 0]) * y2_ref[...]
           + agg_ref[0, :N_NODES] + agg_ref[1, :N_NODES] + b21_ref[...])
    a = jnp.maximum(pre, 0.0)
    h2 = (jnp.dot(a, w22_ref[...], preferred_element_type=jnp.float32)
          + b22_ref[...])
    m = jnp.max(h2, axis=1, keepdims=True)
    lse = jnp.log(jnp.sum(jnp.exp(h2 - m), axis=1, keepdims=True)) + m
    o_ref[...] = h2 - lse


def kernel(x, edge_index, eps1, W11, b11, W12, b12, eps2, W21, b21, W22, b22):
    src = edge_index[0].astype(jnp.int32).reshape(NW, NCHUNK, CH)
    dst = edge_index[1].astype(jnp.int32).reshape(NW, NCHUNK, CH)
    zeros = jnp.zeros((RPT, D_HID), jnp.float32)

    y1 = pl.pallas_call(
        _mm1_body,
        out_shape=jax.ShapeDtypeStruct((N_NODES, D_HID), jnp.float32),
    )(x, W11)

    agg1 = _sc_aggregate(y1, src, dst, zeros)

    emb, y2 = pl.pallas_call(
        _mid_body,
        out_shape=(jax.ShapeDtypeStruct((N_NODES, D_HID), jnp.float32),
                   jax.ShapeDtypeStruct((N_NODES, D_HID), jnp.float32)),
    )(y1, agg1, eps1.reshape(1, 1), b11.reshape(1, D_HID), W12,
      b12.reshape(1, D_HID), W21)

    agg2 = _sc_aggregate(y2, src, dst, zeros)

    out = pl.pallas_call(
        _final_body,
        out_shape=jax.ShapeDtypeStruct((N_NODES, D_HID), jnp.float32),
    )(y2, agg2, eps2.reshape(1, 1), b21.reshape(1, D_HID), W22,
      b22.reshape(1, D_HID))

    return (out, emb)


# speedup vs baseline: 16.1120x; 1.0714x over previous
import functools

import jax
import jax.numpy as jnp
from jax import lax
from jax.experimental import pallas as pl
from jax.experimental.pallas import tpu as pltpu
from jax.experimental.pallas import tpu_sc as plsc

N_NODES = 10000
N_EDGES = 320000
D_IN = 128
D_HID = 64

NC = 2
NS = 16
NW = NC * NS
EPW = N_EDGES // NW
CH = 80
NCHUNK = EPW // CH
NBUF = 8
NGRP = (NCHUNK - 1) // NBUF
N_PAD = 10240
RPT = N_PAD // NS

_sc_mesh = plsc.VectorSubcoreMesh(core_axis_name="c", subcore_axis_name="s")


@functools.partial(
    pl.kernel,
    out_type=jax.ShapeDtypeStruct((NC, N_PAD, D_HID), jnp.float32),
    mesh=_sc_mesh,
    compiler_params=pltpu.CompilerParams(use_tc_tiling_on_sc=False),
    scratch_types=[
        pltpu.VMEM((NCHUNK, CH), jnp.int32),
        pltpu.VMEM((NCHUNK, CH), jnp.int32),
        *[pltpu.VMEM((CH, D_HID), jnp.float32) for _ in range(NBUF)],
        pltpu.VMEM_SHARED((N_PAD, D_HID), jnp.float32),
        *[pltpu.SemaphoreType.DMA for _ in range(2 * NBUF)],
    ],
)
def _sc_aggregate(y_hbm, src_hbm, dst_hbm, zeros_hbm, out_hbm,
                  sidx, didx, *bufs):
    rows = bufs[:NBUF]
    acc = bufs[NBUF]
    gsem = bufs[NBUF + 1:2 * NBUF + 1]
    ssem = bufs[2 * NBUF + 1:]
    cid = lax.axis_index("c")
    sid = lax.axis_index("s")
    wid = cid * NS + sid

    z = pltpu.async_copy(zeros_hbm, acc.at[pl.ds(sid * RPT, RPT)], ssem[0])
    s = pltpu.async_copy(src_hbm.at[wid], sidx, ssem[1])
    t = pltpu.async_copy(dst_hbm.at[wid], didx, ssem[2])
    z.wait()
    s.wait()
    t.wait()
    plsc.subcore_barrier()

    def gather(g, b):
        pltpu.async_copy(y_hbm.at[sidx.at[g]], rows[b], gsem[b])

    def gather_wait(g, b):
        pltpu.make_async_copy(y_hbm.at[sidx.at[g]], rows[b], gsem[b]).wait()

    def scatter(g, b):
        pltpu.async_copy(rows[b], acc.at[didx.at[g]], ssem[b], add=True)

    def scatter_wait(g, b):
        pltpu.make_async_copy(rows[b], acc.at[didx.at[g]], ssem[b]).wait()

    for b in range(NBUF):
        gather(b, b)

    def body(i, carry):
        g = NBUF * i
        for b in range(NBUF):
            gather_wait(g + b, b)
            scatter(g + b, b)
        for b in range(NBUF):
            scatter_wait(g + b, b)

            @pl.when(g + b + NBUF < NCHUNK)
            def _():
                gather(g + b + NBUF, b)

        return carry

    lax.fori_loop(0, NGRP, body, 0)
    tail_start = NGRP * NBUF
    for g in range(tail_start, NCHUNK):
        gather_wait(g, g - tail_start)
        scatter(g, g - tail_start)
    for g in range(tail_start, NCHUNK):
        scatter_wait(g, g - tail_start)

    plsc.subcore_barrier()
    pltpu.sync_copy(acc.at[pl.ds(sid * RPT, RPT)],
                    out_hbm.at[cid, pl.ds(sid * RPT, RPT)])


def _mm1_body(x_ref, w_ref, o_ref):
    o_ref[...] = jnp.dot(x_ref[...], w_ref[...],
                         preferred_element_type=jnp.float32)


def _mid_body(y1_ref, agg_ref, eps_ref, b11_ref, w12_ref, b12_ref, w21_ref,
              emb_ref, y2_ref):
    pre = ((1.0 + eps_ref[0, 0]) * y1_ref[...]
           + agg_ref[0, :N_NODES] + agg_ref[1, :N_NODES] + b11_ref[...])
    h1 = jnp.maximum(pre, 0.0)
    emb = jnp.maximum(
        jnp.dot(h1, w12_ref[...], preferred_element_type=jnp.float32)
        + b12_ref[...], 0.0)
    emb_ref[...] = emb
    y2_ref[...] = jnp.dot(emb, w21_ref[...],
                          preferred_element_type=jnp.float32)


def _final_body(y2_ref, agg_ref, eps_ref, b21_ref, w22_ref, b22_ref, o_ref):
    pre = ((1.0 + eps_ref[0, 0]) * y2_ref[...]
           + agg_ref[0, :N_NODES] + agg_ref[1, :N_NODES] + b21_ref[...])
    a = jnp.maximum(pre, 0.0)
    h2 = (jnp.dot(a, w22_ref[...], preferred_element_type=jnp.float32)
          + b22_ref[...])
    m = jnp.max(h2, axis=1, keepdims=True)
    lse = jnp.log(jnp.sum(jnp.exp(h2 - m), axis=1, keepdims=True)) + m
    o_ref[...] = h2 - lse


def kernel(x, edge_index, eps1, W11, b11, W12, b12, eps2, W21, b21, W22, b22):
    src = edge_index[0].astype(jnp.int32).reshape(NW, NCHUNK, CH)
    dst = edge_index[1].astype(jnp.int32).reshape(NW, NCHUNK, CH)
    zeros = jnp.zeros((RPT, D_HID), jnp.float32)

    y1 = pl.pallas_call(
        _mm1_body,
        out_shape=jax.ShapeDtypeStruct((N_NODES, D_HID), jnp.float32),
    )(x, W11)

    agg1 = _sc_aggregate(y1, src, dst, zeros)

    emb, y2 = pl.pallas_call(
        _mid_body,
        out_shape=(jax.ShapeDtypeStruct((N_NODES, D_HID), jnp.float32),
                   jax.ShapeDtypeStruct((N_NODES, D_HID), jnp.float32)),
    )(y1, agg1, eps1.reshape(1, 1), b11.reshape(1, D_HID), W12,
      b12.reshape(1, D_HID), W21)

    agg2 = _sc_aggregate(y2, src, dst, zeros)

    out = pl.pallas_call(
        _final_body,
        out_shape=jax.ShapeDtypeStruct((N_NODES, D_HID), jnp.float32),
    )(y2, agg2, eps2.reshape(1, 1), b21.reshape(1, D_HID), W22,
      b22.reshape(1, D_HID))

    return (out, emb)


# speedup vs baseline: 16.1935x; 1.0051x over previous
import functools

import jax
import jax.numpy as jnp
from jax import lax
from jax.experimental import pallas as pl
from jax.experimental.pallas import tpu as pltpu
from jax.experimental.pallas import tpu_sc as plsc

N_NODES = 10000
N_EDGES = 320000
D_IN = 128
D_HID = 64

NC = 2
NS = 16
NW = NC * NS
EPW = N_EDGES // NW
CH = 80
NCHUNK = EPW // CH
NBUF = 12
NGRP = (NCHUNK - 1) // NBUF
N_PAD = 10240
RPT = N_PAD // NS

_sc_mesh = plsc.VectorSubcoreMesh(core_axis_name="c", subcore_axis_name="s")


@functools.partial(
    pl.kernel,
    out_type=jax.ShapeDtypeStruct((NC, N_PAD, D_HID), jnp.float32),
    mesh=_sc_mesh,
    compiler_params=pltpu.CompilerParams(use_tc_tiling_on_sc=False),
    scratch_types=[
        pltpu.VMEM((NCHUNK, CH), jnp.int32),
        pltpu.VMEM((NCHUNK, CH), jnp.int32),
        *[pltpu.VMEM((CH, D_HID), jnp.float32) for _ in range(NBUF)],
        pltpu.VMEM_SHARED((N_PAD, D_HID), jnp.float32),
        *[pltpu.SemaphoreType.DMA for _ in range(2 * NBUF)],
    ],
)
def _sc_aggregate(y_hbm, src_hbm, dst_hbm, zeros_hbm, out_hbm,
                  sidx, didx, *bufs):
    rows = bufs[:NBUF]
    acc = bufs[NBUF]
    gsem = bufs[NBUF + 1:2 * NBUF + 1]
    ssem = bufs[2 * NBUF + 1:]
    cid = lax.axis_index("c")
    sid = lax.axis_index("s")
    wid = cid * NS + sid

    z = pltpu.async_copy(zeros_hbm, acc.at[pl.ds(sid * RPT, RPT)], ssem[0])
    s = pltpu.async_copy(src_hbm.at[wid], sidx, ssem[1])
    t = pltpu.async_copy(dst_hbm.at[wid], didx, ssem[2])
    z.wait()
    s.wait()
    t.wait()
    plsc.subcore_barrier()

    def gather(g, b):
        pltpu.async_copy(y_hbm.at[sidx.at[g]], rows[b], gsem[b])

    def gather_wait(g, b):
        pltpu.make_async_copy(y_hbm.at[sidx.at[g]], rows[b], gsem[b]).wait()

    def scatter(g, b):
        pltpu.async_copy(rows[b], acc.at[didx.at[g]], ssem[b], add=True)

    def scatter_wait(g, b):
        pltpu.make_async_copy(rows[b], acc.at[didx.at[g]], ssem[b]).wait()

    for b in range(NBUF):
        gather(b, b)

    def body(i, carry):
        g = NBUF * i
        for b in range(NBUF):
            gather_wait(g + b, b)
            scatter(g + b, b)
        for b in range(NBUF):
            scatter_wait(g + b, b)

            @pl.when(g + b + NBUF < NCHUNK)
            def _():
                gather(g + b + NBUF, b)

        return carry

    lax.fori_loop(0, NGRP, body, 0)
    tail_start = NGRP * NBUF
    for g in range(tail_start, NCHUNK):
        gather_wait(g, g - tail_start)
        scatter(g, g - tail_start)
    for g in range(tail_start, NCHUNK):
        scatter_wait(g, g - tail_start)

    plsc.subcore_barrier()
    pltpu.sync_copy(acc.at[pl.ds(sid * RPT, RPT)],
                    out_hbm.at[cid, pl.ds(sid * RPT, RPT)])


def _mm1_body(x_ref, w_ref, o_ref):
    o_ref[...] = jnp.dot(x_ref[...], w_ref[...],
                         preferred_element_type=jnp.float32)


def _mid_body(y1_ref, agg_ref, eps_ref, b11_ref, w12_ref, b12_ref, w21_ref,
              emb_ref, y2_ref):
    pre = ((1.0 + eps_ref[0, 0]) * y1_ref[...]
           + agg_ref[0, :N_NODES] + agg_ref[1, :N_NODES] + b11_ref[...])
    h1 = jnp.maximum(pre, 0.0)
    emb = jnp.maximum(
        jnp.dot(h1, w12_ref[...], preferred_element_type=jnp.float32)
        + b12_ref[...], 0.0)
    emb_ref[...] = emb
    y2_ref[...] = jnp.dot(emb, w21_ref[...],
                          preferred_element_type=jnp.float32)


def _final_body(y2_ref, agg_ref, eps_ref, b21_ref, w22_ref, b22_ref, o_ref):
    pre = ((1.0 + eps_ref[0, 0]) * y2_ref[...]
           + agg_ref[0, :N_NODES] + agg_ref[1, :N_NODES] + b21_ref[...])
    a = jnp.maximum(pre, 0.0)
    h2 = (jnp.dot(a, w22_ref[...], preferred_element_type=jnp.float32)
          + b22_ref[...])
    m = jnp.max(h2, axis=1, keepdims=True)
    lse = jnp.log(jnp.sum(jnp.exp(h2 - m), axis=1, keepdims=True)) + m
    o_ref[...] = h2 - lse


def kernel(x, edge_index, eps1, W11, b11, W12, b12, eps2, W21, b21, W22, b22):
    src = edge_index[0].astype(jnp.int32).reshape(NW, NCHUNK, CH)
    dst = edge_index[1].astype(jnp.int32).reshape(NW, NCHUNK, CH)
    zeros = jnp.zeros((RPT, D_HID), jnp.float32)

    y1 = pl.pallas_call(
        _mm1_body,
        out_shape=jax.ShapeDtypeStruct((N_NODES, D_HID), jnp.float32),
    )(x, W11)

    agg1 = _sc_aggregate(y1, src, dst, zeros)

    emb, y2 = pl.pallas_call(
        _mid_body,
        out_shape=(jax.ShapeDtypeStruct((N_NODES, D_HID), jnp.float32),
                   jax.ShapeDtypeStruct((N_NODES, D_HID), jnp.float32)),
    )(y1, agg1, eps1.reshape(1, 1), b11.reshape(1, D_HID), W12,
      b12.reshape(1, D_HID), W21)

    agg2 = _sc_aggregate(y2, src, dst, zeros)

    out = pl.pallas_call(
        _final_body,
        out_shape=jax.ShapeDtypeStruct((N_NODES, D_HID), jnp.float32),
    )(y2, agg2, eps2.reshape(1, 1), b21.reshape(1, D_HID), W22,
      b22.reshape(1, D_HID))

    return (out, emb)
